# baseline (speedup 1.0000x reference)
import functools

import jax
import jax.numpy as jnp
from jax import lax
from jax.experimental import pallas as pl
from jax.experimental.pallas import tpu as pltpu
from jax.experimental.pallas import tpu_sc as plsc

NUM_CLASSES = 1000
N_CTX = 16
CTX_DIM = 512
SEQ_LEN = 77
SUFFIX_LEN = SEQ_LEN - 1 - N_CTX
PAD_SEQ = 80

NC = 2
NS = 16
NW = NC * NS


def _make_sc_kernel(B: int):
    b_per_w = B // NW
    mesh = plsc.VectorSubcoreMesh(
        core_axis_name="c", subcore_axis_name="s", num_cores=NC, num_subcores=NS
    )

    @functools.partial(
        pl.kernel,
        out_type=jax.ShapeDtypeStruct((B, SEQ_LEN, CTX_DIM), jnp.float32),
        mesh=mesh,
        compiler_params=pltpu.CompilerParams(
            needs_layout_passes=False, use_tc_tiling_on_sc=False
        ),
        scratch_types=[
            pltpu.VMEM((1, b_per_w), jnp.int32),
            pltpu.VMEM((16,), jnp.int32),
            pltpu.VMEM((16,), jnp.int32),
            pltpu.VMEM((1, PAD_SEQ, CTX_DIM), jnp.float32),
            pltpu.VMEM((1, PAD_SEQ, CTX_DIM), jnp.float32),
            pltpu.VMEM((1, 5, CTX_DIM), jnp.float32),
            pltpu.VMEM((1, 5, CTX_DIM), jnp.float32),
            pltpu.SemaphoreType.DMA,
            pltpu.SemaphoreType.DMA,
            pltpu.SemaphoreType.DMA,
            pltpu.SemaphoreType.DMA,
        ],
    )
    def body(label_hbm, table_hbm, out_hbm,
             idx_v, is0, is1, row0, row1, tail0, tail1,
             gsem0, gsem1, wsem0, wsem1):
        tail_v = (tail0, tail1)
        idx_s = (is0, is1)
        row_v = (row0, row1)
        gsem = (gsem0, gsem1)
        wsem = (wsem0, wsem1)
        wid = lax.axis_index("s") * NC + lax.axis_index("c")
        base = wid * b_per_w
        pltpu.sync_copy(label_hbm.at[wid], idx_v)
        zeros16 = jnp.zeros((16,), jnp.int32)

        def stage_idx(i, k):
            ivec = plsc.load_gather(
                idx_v, [zeros16, jnp.full((16,), i, jnp.int32)]
            )
            idx_s[k][...] = ivec

        def g_copy(k):
            return pltpu.make_async_copy(
                table_hbm.at[idx_s[k].at[pl.ds(0, 1)]], row_v[k], gsem[k]
            )

        def w_copy(i, k):
            return pltpu.make_async_copy(
                row_v[k].at[:, pl.ds(0, 72)],
                out_hbm.at[pl.ds(base + i, 1), pl.ds(0, 72)], wsem[k]
            )

        def t_copy(i, k):
            return pltpu.make_async_copy(
                tail_v[k],
                out_hbm.at[pl.ds(base + i, 1), pl.ds(72, 5)], wsem[k]
            )

        def fill_tail(k):
            for r in range(5):
                for t in range(CTX_DIM // 16):
                    tail_v[k][0, r, pl.ds(16 * t, 16)] = (
                        row_v[k][0, 72 + r, pl.ds(16 * t, 16)]
                    )

        for k in (0, 1):
            stage_idx(k, k)
            g_copy(k).start()

        def pair(g, _):
            for k in (0, 1):
                i = 2 * g + k
                g_copy(k).wait()
                wc = w_copy(i, k)
                wc.start()
                fill_tail(k)
                tc = t_copy(i, k)
                tc.start()
                wc.wait()
                tc.wait()

                @pl.when(i + 2 < b_per_w)
                def _():
                    stage_idx(i + 2, k)
                    g_copy(k).start()

            return 0

        lax.fori_loop(0, b_per_w // 2, pair, 0)

    return body


def kernel(label, cls_ctx, token_prefix, token_suffix):
    B = label.shape[0]
    table = (
        jnp.pad(jnp.broadcast_to(token_prefix, (NUM_CLASSES, 1, CTX_DIM)),
                ((0, 0), (0, PAD_SEQ - 1), (0, 0)))
        + jnp.pad(cls_ctx, ((0, 0), (1, PAD_SEQ - 1 - N_CTX), (0, 0)))
        + jnp.pad(token_suffix,
                  ((0, 0), (1 + N_CTX, PAD_SEQ - SEQ_LEN), (0, 0)))
    )
    label3 = label.astype(jnp.int32).reshape(NW, 1, B // NW)
    return _make_sc_kernel(B)(label3, table)


# speedup vs baseline: 1.5242x; 1.5242x over previous
import functools

import jax
import jax.numpy as jnp
from jax import lax
from jax.experimental import pallas as pl
from jax.experimental.pallas import tpu as pltpu
from jax.experimental.pallas import tpu_sc as plsc

NUM_CLASSES = 1000
N_CTX = 16
CTX_DIM = 512
SEQ_LEN = 77
SUFFIX_LEN = SEQ_LEN - 1 - N_CTX
PAD_SEQ = 80

NC = 2
NS = 16
NW = NC * NS


def _make_sc_kernel(B: int):
    b_per_w = B // NW
    mesh = plsc.VectorSubcoreMesh(
        core_axis_name="c", subcore_axis_name="s", num_cores=NC, num_subcores=NS
    )

    @functools.partial(
        pl.kernel,
        out_type=jax.ShapeDtypeStruct((B, SEQ_LEN, CTX_DIM), jnp.float32),
        mesh=mesh,
        compiler_params=pltpu.CompilerParams(needs_layout_passes=False),
        scratch_types=[
            pltpu.VMEM((1, b_per_w), jnp.int32),
            pltpu.VMEM((16,), jnp.int32),
            pltpu.VMEM((16,), jnp.int32),
            pltpu.VMEM((1, PAD_SEQ, CTX_DIM), jnp.float32),
            pltpu.VMEM((1, PAD_SEQ, CTX_DIM), jnp.float32),
            pltpu.VMEM((1, 5, CTX_DIM), jnp.float32),
            pltpu.VMEM((1, 5, CTX_DIM), jnp.float32),
            pltpu.SemaphoreType.DMA,
            pltpu.SemaphoreType.DMA,
            pltpu.SemaphoreType.DMA,
            pltpu.SemaphoreType.DMA,
        ],
    )
    def body(label_hbm, table_hbm, out_hbm,
             idx_v, is0, is1, row0, row1, tail0, tail1,
             gsem0, gsem1, wsem0, wsem1):
        tail_v = (tail0, tail1)
        idx_s = (is0, is1)
        row_v = (row0, row1)
        gsem = (gsem0, gsem1)
        wsem = (wsem0, wsem1)
        wid = lax.axis_index("s") * NC + lax.axis_index("c")
        base = wid * b_per_w
        pltpu.sync_copy(label_hbm.at[wid], idx_v)
        zeros16 = jnp.zeros((16,), jnp.int32)

        def stage_idx(i, k):
            ivec = plsc.load_gather(
                idx_v, [zeros16, jnp.full((16,), i, jnp.int32)]
            )
            idx_s[k][...] = ivec

        def g_copy(k):
            return pltpu.make_async_copy(
                table_hbm.at[idx_s[k].at[pl.ds(0, 1)]], row_v[k], gsem[k]
            )

        def w_copy(i, k):
            return pltpu.make_async_copy(
                row_v[k].at[:, pl.ds(0, 72)],
                out_hbm.at[pl.ds(base + i, 1), pl.ds(0, 72)], wsem[k]
            )

        def t_copy(i, k):
            return pltpu.make_async_copy(
                tail_v[k],
                out_hbm.at[pl.ds(base + i, 1), pl.ds(72, 5)], wsem[k]
            )

        def fill_tail(k):
            for r in range(5):
                for t in range(CTX_DIM // 16):
                    tail_v[k][0, r, pl.ds(16 * t, 16)] = (
                        row_v[k][0, 72 + r, pl.ds(16 * t, 16)]
                    )

        for k in (0, 1):
            stage_idx(k, k)
            g_copy(k).start()

        def pair(g, _):
            for k in (0, 1):
                i = 2 * g + k
                g_copy(k).wait()
                wc = w_copy(i, k)
                wc.start()
                fill_tail(k)
                tc = t_copy(i, k)
                tc.start()
                wc.wait()
                tc.wait()

                @pl.when(i + 2 < b_per_w)
                def _():
                    stage_idx(i + 2, k)
                    g_copy(k).start()

            return 0

        lax.fori_loop(0, b_per_w // 2, pair, 0)

    return body


def kernel(label, cls_ctx, token_prefix, token_suffix):
    B = label.shape[0]
    table = jnp.concatenate(
        [
            jnp.broadcast_to(token_prefix, (NUM_CLASSES, 1, CTX_DIM)),
            cls_ctx,
            token_suffix,
            jnp.zeros((NUM_CLASSES, PAD_SEQ - SEQ_LEN, CTX_DIM), jnp.float32),
        ],
        axis=1,
    )
    label3 = label.astype(jnp.int32).reshape(NW, 1, B // NW)
    return _make_sc_kernel(B)(label3, table)
